# baseline (speedup 1.0000x reference)
import functools

import jax
import jax.numpy as jnp
from jax.experimental import pallas as pl
from jax.experimental.pallas import tpu as pltpu

_BATCH, _SEQ, _D = 4, 4096, 2048
_E = 64
_TOKENS = _BATCH * _SEQ
_TB = 512
_HALF = _TOKENS // 2


def _top2_block(logits):
    tb = logits.shape[0]
    iota = jax.lax.broadcasted_iota(jnp.int32, (tb, _E), 1)

    m1 = jnp.max(logits, axis=-1, keepdims=True)
    i1 = jnp.min(jnp.where(logits == m1, iota, _E), axis=-1, keepdims=True)
    masked = jnp.where(iota == i1, -jnp.inf, logits)
    m2 = jnp.max(masked, axis=-1, keepdims=True)
    i2 = jnp.min(jnp.where(masked == m2, iota, _E), axis=-1, keepdims=True)

    e = jnp.exp(m2 - m1)
    denom = 1.0 + e
    w = jnp.concatenate([1.0 / denom, e / denom], axis=1)
    idx = jnp.concatenate([i1, i2], axis=1)

    en = jnp.exp(logits - m1)
    r = 1.0 / jnp.sum(en, axis=-1, keepdims=True)
    part = jnp.dot(r.reshape(1, tb), en, preferred_element_type=jnp.float32)
    return w, idx, part


def _router_kernel(x0_ref, x1_ref, wt_ref, b_ref, w_out_ref, i_out_ref,
                   loss_ref, acc_ref, *, n_steps, n_tokens):
    step = pl.program_id(0)
    bias = b_ref[...]

    l0 = jnp.dot(x0_ref[...], wt_ref[...],
                 preferred_element_type=jnp.float32) + bias
    w0, idx0, part0 = _top2_block(l0)
    l1 = jnp.dot(x1_ref[...], wt_ref[...],
                 preferred_element_type=jnp.float32) + bias
    w1, idx1, part1 = _top2_block(l1)

    w_out_ref[0, :, :] = w0
    w_out_ref[1, :, :] = w1
    i_out_ref[0, :, :] = idx0
    i_out_ref[1, :, :] = idx1

    @pl.when(step == 0)
    def _():
        acc_ref[...] = jnp.zeros_like(acc_ref)

    acc_ref[...] += part0 + part1

    @pl.when(step == n_steps - 1)
    def _():
        usage = acc_ref[...] * (1.0 / n_tokens)
        ssq = jnp.sum(usage * usage, axis=1, keepdims=True)
        loss_ref[...] = _E * ssq - 1.0


def kernel(x, gate_w, gate_b):
    xf = x.reshape(_TOKENS, _D)
    wt = gate_w.T
    b2 = gate_b.reshape(1, _E)
    n_steps = _HALF // _TB
    half_blocks = _HALF // _TB

    weights, indices, loss = pl.pallas_call(
        functools.partial(_router_kernel, n_steps=n_steps, n_tokens=_TOKENS),
        grid=(n_steps,),
        in_specs=[
            pl.BlockSpec((_TB, _D), lambda i: (i, 0)),
            pl.BlockSpec((_TB, _D), lambda i: (i + half_blocks, 0)),
            pl.BlockSpec((_D, _E), lambda i: (0, 0)),
            pl.BlockSpec((1, _E), lambda i: (0, 0)),
        ],
        out_specs=[
            pl.BlockSpec((2, _TB, 2), lambda i: (0, i, 0)),
            pl.BlockSpec((2, _TB, 2), lambda i: (0, i, 0)),
            pl.BlockSpec((1, 1), lambda i: (0, 0)),
        ],
        out_shape=[
            jax.ShapeDtypeStruct((2, _HALF, 2), jnp.float32),
            jax.ShapeDtypeStruct((2, _HALF, 2), jnp.int32),
            jax.ShapeDtypeStruct((1, 1), jnp.float32),
        ],
        scratch_shapes=[pltpu.VMEM((1, _E), jnp.float32)],
    )(xf, xf, wt, b2)

    return (weights.reshape(_BATCH, _SEQ, 2),
            indices.reshape(_BATCH, _SEQ, 2),
            loss[0, 0])


# speedup vs baseline: 1.1092x; 1.1092x over previous
import functools

import jax
import jax.numpy as jnp
from jax.experimental import pallas as pl
from jax.experimental.pallas import tpu as pltpu

_BATCH, _SEQ, _D = 4, 4096, 2048
_E = 64
_TOKENS = _BATCH * _SEQ
_TB = 2048


def _router_kernel(x_ref, w_ref, b_ref, w_out_ref, i_out_ref, loss_ref,
                   acc_ref, *, n_steps, n_tokens):
    step = pl.program_id(0)

    lt = jax.lax.dot_general(
        w_ref[...], x_ref[...],
        (((1,), (1,)), ((), ())),
        preferred_element_type=jnp.float32) + b_ref[...]

    tb = lt.shape[1]
    iota = jax.lax.broadcasted_iota(jnp.int32, (_E, tb), 0)

    m1 = jnp.max(lt, axis=0, keepdims=True)
    i1 = jnp.min(jnp.where(lt == m1, iota, _E), axis=0, keepdims=True)
    masked = jnp.where(iota == i1, -jnp.inf, lt)
    m2 = jnp.max(masked, axis=0, keepdims=True)
    i2 = jnp.min(jnp.where(masked == m2, iota, _E), axis=0, keepdims=True)

    e = jnp.exp(m2 - m1)
    denom = 1.0 + e
    w_out_ref[...] = jnp.concatenate([1.0 / denom, e / denom], axis=0).T
    i_out_ref[...] = jnp.concatenate([i1, i2], axis=0).T

    en = jnp.exp(lt - m1)
    r = (1.0 / jnp.sum(en, axis=0, keepdims=True)).T
    part = jnp.dot(en, r, preferred_element_type=jnp.float32)

    @pl.when(step == 0)
    def _():
        acc_ref[...] = jnp.zeros_like(acc_ref)

    acc_ref[...] += part

    @pl.when(step == n_steps - 1)
    def _():
        usage = acc_ref[...] * (1.0 / n_tokens)
        ssq = jnp.sum(usage * usage, axis=0, keepdims=True)
        loss_ref[...] = _E * ssq - 1.0


def kernel(x, gate_w, gate_b):
    xf = x.reshape(_TOKENS, _D)
    b2 = gate_b.reshape(_E, 1)
    n_steps = _TOKENS // _TB

    weights, indices, loss = pl.pallas_call(
        functools.partial(_router_kernel, n_steps=n_steps, n_tokens=_TOKENS),
        grid=(n_steps,),
        in_specs=[
            pl.BlockSpec((_TB, _D), lambda i: (i, 0)),
            pl.BlockSpec((_E, _D), lambda i: (0, 0)),
            pl.BlockSpec((_E, 1), lambda i: (0, 0)),
        ],
        out_specs=[
            pl.BlockSpec((_TB, 2), lambda i: (i, 0)),
            pl.BlockSpec((_TB, 2), lambda i: (i, 0)),
            pl.BlockSpec((1, 1), lambda i: (0, 0)),
        ],
        out_shape=[
            jax.ShapeDtypeStruct((_TOKENS, 2), jnp.float32),
            jax.ShapeDtypeStruct((_TOKENS, 2), jnp.int32),
            jax.ShapeDtypeStruct((1, 1), jnp.float32),
        ],
        scratch_shapes=[pltpu.VMEM((_E, 1), jnp.float32)],
    )(xf, gate_w, b2)

    return (weights.reshape(_BATCH, _SEQ, 2),
            indices.reshape(_BATCH, _SEQ, 2),
            loss[0, 0])
